# baseline (speedup 1.0000x reference)
import functools

import jax
import jax.numpy as jnp
from jax import lax
from jax.experimental import pallas as pl
from jax.experimental.pallas import tpu as pltpu
from jax.experimental.pallas import tpu_sc as plsc

NC = 2
NS = 16
L = 16
NW = NC * NS
B = 16384
D = 32
BPW = B // NW
NCHUNK = 4
CB = BPW // NCHUNK
NGROUP = BPW // L


def _sqrt16(x):
    y = plsc.bitcast(jnp.int32(0x5F3759DF) - (plsc.bitcast(x, jnp.int32) >> 1),
                     jnp.float32)
    for _ in range(4):
        y = y * (jnp.float32(1.5) - jnp.float32(0.5) * x * y * y)
    return x * y


def _tec_body(s_hbm, r_hbm, o_hbm, ent_hbm, rel_hbm, out_hbm,
              sidx, ridx, oidx, srows, rrows, orows, h, score, sem):
    wid = lax.axis_index("s") * NC + lax.axis_index("c")
    base = wid * BPW

    for j in range(NCHUNK):
        off = base + j * CB
        pltpu.sync_copy(s_hbm.at[pl.ds(off, CB)], sidx.at[j])
        pltpu.sync_copy(r_hbm.at[pl.ds(off, CB)], ridx.at[j])
        pltpu.sync_copy(o_hbm.at[pl.ds(off, CB)], oidx.at[j])

    copies = []
    for j in range(NCHUNK):
        copies.append(pltpu.async_copy(ent_hbm.at[sidx.at[j]], srows.at[j], sem))
        copies.append(pltpu.async_copy(ent_hbm.at[oidx.at[j]], orows.at[j], sem))
        copies.append(pltpu.async_copy(rel_hbm.at[ridx.at[j]], rrows.at[j], sem))
    for c in copies:
        c.wait()

    for j in range(NCHUNK):
        def ebody(i, _, j=j):
            a0 = srows[j, i, pl.ds(0, L)] + rrows[j, i, pl.ds(0, L)] - orows[j, i, pl.ds(0, L)]
            a1 = srows[j, i, pl.ds(L, L)] + rrows[j, i, pl.ds(L, L)] - orows[j, i, pl.ds(L, L)]
            h[pl.ds(pl.multiple_of((j * CB + i) * L, L), L)] = a0 * a0 + a1 * a1
            return 0
        lax.fori_loop(0, CB, ebody, 0)

    lane16 = lax.iota(jnp.int32, L) * L

    def rbody(g, _):
        base_i = g * (L * L) + lane16
        acc = plsc.load_gather(h, [base_i])
        for k in range(1, L):
            acc = acc + plsc.load_gather(h, [base_i + k])
        score[pl.ds(pl.multiple_of(g * L, L), L)] = _sqrt16(acc)
        return 0

    lax.fori_loop(0, NGROUP, rbody, 0)

    pltpu.sync_copy(score, out_hbm.at[pl.ds(base, BPW)])


def kernel(s, r, o, entity_embeddings, relation_embeddings):
    mesh = plsc.VectorSubcoreMesh(
        core_axis_name="c", subcore_axis_name="s",
        num_cores=NC, num_subcores=NS)
    run = functools.partial(
        pl.kernel,
        out_type=jax.ShapeDtypeStruct((B,), jnp.float32),
        mesh=mesh,
        compiler_params=pltpu.CompilerParams(
            needs_layout_passes=False, use_tc_tiling_on_sc=False),
        scratch_types=[
            pltpu.VMEM((NCHUNK, CB), jnp.int32),
            pltpu.VMEM((NCHUNK, CB), jnp.int32),
            pltpu.VMEM((NCHUNK, CB), jnp.int32),
            pltpu.VMEM((NCHUNK, CB, D), jnp.float32),
            pltpu.VMEM((NCHUNK, CB, D), jnp.float32),
            pltpu.VMEM((NCHUNK, CB, D), jnp.float32),
            pltpu.VMEM((BPW * L,), jnp.float32),
            pltpu.VMEM((BPW,), jnp.float32),
            pltpu.SemaphoreType.DMA,
        ],
    )(_tec_body)
    return run(s, r, o, entity_embeddings, relation_embeddings)
